# baseline (speedup 1.0000x reference)
import jax
import jax.numpy as jnp
from jax.experimental import pallas as pl
from jax.experimental.pallas import tpu as pltpu

N = 128
NEG = 1e8
TI = 64


def _leaky(x):
    return jnp.maximum(x, 0.01 * x)


def _softmax_rows(x):
    m = jnp.max(x, axis=-1, keepdims=True)
    e = jnp.exp(x - m)
    return e / jnp.sum(e, axis=-1, keepdims=True)


def _DOT(a, b):
    return jnp.dot(a.astype(jnp.bfloat16), b.astype(jnp.bfloat16),
                   preferred_element_type=jnp.float32)


def _edge_sim(x, w0t, w1t, w2t, woutt, b, sp_ref, f_ref, layer):
    D = x.shape[1]
    off = 0
    for t in range(N // TI):
        i0 = t * TI
        jl = N - i0
        rows = TI * jl
        if layer == 0:
            xt = x[i0:i0 + TI]
            xj = x[i0:]
            f = jnp.exp(-jnp.abs(xt[:, None, :] - xj[None, :, :]))
            fb = f.reshape(rows, D).astype(jnp.bfloat16)
            f_ref[off:off + rows] = fb
        else:
            xt = x[i0:i0 + TI, 128:]
            xj = x[i0:, 128:]
            f = jnp.exp(-jnp.abs(xt[:, None, :] - xj[None, :, :]))
            fb = jnp.concatenate(
                [f_ref[off:off + rows],
                 f.reshape(rows, 128).astype(jnp.bfloat16)], axis=-1)
        h = _leaky(_DOT(fb, w0t))
        h = _leaky(_DOT(h, w1t))
        h = _leaky(_DOT(h, w2t))
        s = _DOT(h, woutt).reshape(TI, jl) + b
        sp_ref[i0:i0 + TI, i0:] = s
        sp_ref[i0:, i0:i0 + TI] = s.T
        off += rows


def _mask_aggr(sp, x, nw0t):
    col = jax.lax.broadcasted_iota(jnp.int32, (N, N), 1)
    row = jax.lax.broadcasted_iota(jnp.int32, (N, N), 0)
    eyef = (row == col).astype(jnp.float32)
    sim = _softmax_rows(sp - eyef * NEG)
    dsim = _softmax_rows((1.0 - sim) - eyef * NEG)

    wt = jnp.concatenate([sim.T, dsim.T], axis=1)
    rowj = jax.lax.broadcasted_iota(jnp.int32, (N, 2 * N), 0)
    mt = jnp.zeros((N, 2 * N), jnp.float32)
    for _ in range(16):
        mx = jnp.max(wt, axis=0, keepdims=True)
        idx = jnp.min(jnp.where(wt == mx, rowj, N), axis=0, keepdims=True)
        onehot = rowj == idx
        mt = jnp.where(onehot, 1.0, mt)
        wt = jnp.where(onehot, -1e30, wt)

    m0t = mt[:, :N]
    m1t = mt[:, N:]
    s0 = ((m0t.T + m0t) > 0).astype(jnp.float32)
    s1 = ((m1t.T + m1t) > 0).astype(jnp.float32)
    a0 = _softmax_rows(sim - (1.0 - s0) * NEG)
    a1 = _softmax_rows(dsim - (1.0 - s1) * NEG)
    dm = 1.0 - eyef
    ef0 = a0 * dm
    ef1 = a1 * dm
    ef0 = ef0 / jnp.maximum(jnp.sum(jnp.abs(ef0), -1, keepdims=True), 1e-12)
    ef1 = ef1 / jnp.maximum(jnp.sum(jnp.abs(ef1), -1, keepdims=True), 1e-12)
    ag0 = _DOT(ef0, x)
    ag1 = _DOT(ef1, x)
    nf = jnp.concatenate([x, ag0, ag1], axis=-1)
    return _leaky(_DOT(nf, nw0t))


def _mega_body(x_ref, e0w0_ref, e0w1_ref, e0w2_ref, e0wo_ref, e0b_ref,
               n0_ref, e1w0_ref, e1w1_ref, e1w2_ref, e1wo_ref, e1b_ref,
               n1_ref, fc1_ref, b1_ref, fc2_ref, b2_ref, out_ref, sp_ref,
               f_ref):
    x0 = x_ref[0]
    _edge_sim(x0, e0w0_ref[...], e0w1_ref[...], e0w2_ref[...], e0wo_ref[...],
              e0b_ref[0, 0], sp_ref, f_ref, 0)
    nf0 = _mask_aggr(sp_ref[...], x0, n0_ref[...])
    x1 = jnp.concatenate([x0, nf0], axis=-1)
    _edge_sim(x1, e1w0_ref[...], e1w1_ref[...], e1w2_ref[...], e1wo_ref[...],
              e1b_ref[0, 0], sp_ref, f_ref, 1)
    nf1 = _mask_aggr(sp_ref[...], x1, n1_ref[...])
    x2 = jnp.concatenate([x1, nf1], axis=-1)
    h = _leaky(_DOT(x2, fc1_ref[...]) + b1_ref[...])
    out_ref[0] = _DOT(h, fc2_ref[...]) + b2_ref[...]


def kernel(all_emb, e0_w0, e0_w1, e0_w2, e0_wout, e0_bout, n0_w0, e1_w0,
           e1_w1, e1_w2, e1_wout, e1_bout, n1_w0, fc1_w, fc1_b, fc2_w,
           fc2_b):
    B = all_emb.shape[0]
    ws = [e0_w0.T, e0_w1.T, e0_w2.T, e0_wout.T, e0_bout.reshape(1, 1),
          n0_w0.T, e1_w0.T, e1_w1.T, e1_w2.T, e1_wout.T,
          e1_bout.reshape(1, 1), n1_w0.T, fc1_w.T, fc1_b.reshape(1, -1),
          fc2_w.T, fc2_b.reshape(1, -1)]
    full = lambda a: pl.BlockSpec(a.shape, lambda b: (0,) * a.ndim)
    return pl.pallas_call(
        _mega_body,
        grid=(B,),
        in_specs=[pl.BlockSpec((1, N, 128), lambda b: (b, 0, 0))]
        + [full(w) for w in ws],
        out_specs=pl.BlockSpec((1, N, 2), lambda b: (b, 0, 0)),
        out_shape=jax.ShapeDtypeStruct((B, N, 2), jnp.float32),
        scratch_shapes=[pltpu.VMEM((N, N), jnp.float32),
                        pltpu.VMEM((TI * (N + TI) * (N // TI) // 2, 128),
                                   jnp.bfloat16)],
        compiler_params=pltpu.CompilerParams(
            dimension_semantics=("parallel",)),
    )(all_emb, *ws)


# speedup vs baseline: 1.0585x; 1.0585x over previous
import jax
import jax.numpy as jnp
from jax.experimental import pallas as pl
from jax.experimental.pallas import tpu as pltpu

N = 128
NEG = 1e8
TI = 16


def _leaky(x):
    return jnp.maximum(x, 0.01 * x)


def _softmax_rows(x):
    m = jnp.max(x, axis=-1, keepdims=True)
    e = jnp.exp(x - m)
    return e / jnp.sum(e, axis=-1, keepdims=True)


def _DOT(a, b):
    return jnp.dot(a.astype(jnp.bfloat16), b.astype(jnp.bfloat16),
                   preferred_element_type=jnp.float32)


def _edge_sim(x, w0t, w1t, w2t, woutt, b, sp_ref, f_ref, layer):
    D = x.shape[1]
    off = 0
    for t in range(N // TI):
        i0 = t * TI
        jl = N - i0
        rows = TI * jl
        if layer == 0:
            xt = x[i0:i0 + TI]
            xj = x[i0:]
            f = jnp.exp(-jnp.abs(xt[:, None, :] - xj[None, :, :]))
            fb = f.reshape(rows, D).astype(jnp.bfloat16)
            f_ref[off:off + rows] = fb
        else:
            xt = x[i0:i0 + TI, 128:]
            xj = x[i0:, 128:]
            f = jnp.exp(-jnp.abs(xt[:, None, :] - xj[None, :, :]))
            fb = jnp.concatenate(
                [f_ref[off:off + rows],
                 f.reshape(rows, 128).astype(jnp.bfloat16)], axis=-1)
        h = _leaky(_DOT(fb, w0t))
        h = _leaky(_DOT(h, w1t))
        h = _leaky(_DOT(h, w2t))
        s = _DOT(h, woutt).reshape(TI, jl) + b
        sp_ref[i0:i0 + TI, i0:] = s
        sp_ref[i0:, i0:i0 + TI] = s.T
        off += rows


def _mask_aggr(sp, x, nw0t):
    col = jax.lax.broadcasted_iota(jnp.int32, (N, N), 1)
    row = jax.lax.broadcasted_iota(jnp.int32, (N, N), 0)
    eyef = (row == col).astype(jnp.float32)
    sim = _softmax_rows(sp - eyef * NEG)
    dsim = _softmax_rows((1.0 - sim) - eyef * NEG)

    wt = jnp.concatenate([sim.T, dsim.T], axis=1)
    rowj = jax.lax.broadcasted_iota(jnp.int32, (N, 2 * N), 0)
    mt = jnp.zeros((N, 2 * N), jnp.float32)
    for _ in range(16):
        mx = jnp.max(wt, axis=0, keepdims=True)
        idx = jnp.min(jnp.where(wt == mx, rowj, N), axis=0, keepdims=True)
        onehot = rowj == idx
        mt = jnp.where(onehot, 1.0, mt)
        wt = jnp.where(onehot, -1e30, wt)

    m0t = mt[:, :N]
    m1t = mt[:, N:]
    s0 = ((m0t.T + m0t) > 0).astype(jnp.float32)
    s1 = ((m1t.T + m1t) > 0).astype(jnp.float32)
    a0 = _softmax_rows(sim - (1.0 - s0) * NEG)
    a1 = _softmax_rows(dsim - (1.0 - s1) * NEG)
    dm = 1.0 - eyef
    ef0 = a0 * dm
    ef1 = a1 * dm
    ef0 = ef0 / jnp.maximum(jnp.sum(jnp.abs(ef0), -1, keepdims=True), 1e-12)
    ef1 = ef1 / jnp.maximum(jnp.sum(jnp.abs(ef1), -1, keepdims=True), 1e-12)
    ag0 = _DOT(ef0, x)
    ag1 = _DOT(ef1, x)
    nf = jnp.concatenate([x, ag0, ag1], axis=-1)
    return _leaky(_DOT(nf, nw0t))


def _mega_body(x_ref, e0w0_ref, e0w1_ref, e0w2_ref, e0wo_ref, e0b_ref,
               n0_ref, e1w0_ref, e1w1_ref, e1w2_ref, e1wo_ref, e1b_ref,
               n1_ref, fc1_ref, b1_ref, fc2_ref, b2_ref, out_ref, sp_ref,
               f_ref):
    x0 = x_ref[0]
    _edge_sim(x0, e0w0_ref[...], e0w1_ref[...], e0w2_ref[...], e0wo_ref[...],
              e0b_ref[0, 0], sp_ref, f_ref, 0)
    nf0 = _mask_aggr(sp_ref[...], x0, n0_ref[...])
    x1 = jnp.concatenate([x0, nf0], axis=-1)
    _edge_sim(x1, e1w0_ref[...], e1w1_ref[...], e1w2_ref[...], e1wo_ref[...],
              e1b_ref[0, 0], sp_ref, f_ref, 1)
    nf1 = _mask_aggr(sp_ref[...], x1, n1_ref[...])
    x2 = jnp.concatenate([x1, nf1], axis=-1)
    h = _leaky(_DOT(x2, fc1_ref[...]) + b1_ref[...])
    out_ref[0] = _DOT(h, fc2_ref[...]) + b2_ref[...]


def kernel(all_emb, e0_w0, e0_w1, e0_w2, e0_wout, e0_bout, n0_w0, e1_w0,
           e1_w1, e1_w2, e1_wout, e1_bout, n1_w0, fc1_w, fc1_b, fc2_w,
           fc2_b):
    B = all_emb.shape[0]
    ws = [e0_w0.T, e0_w1.T, e0_w2.T, e0_wout.T, e0_bout.reshape(1, 1),
          n0_w0.T, e1_w0.T, e1_w1.T, e1_w2.T, e1_wout.T,
          e1_bout.reshape(1, 1), n1_w0.T, fc1_w.T, fc1_b.reshape(1, -1),
          fc2_w.T, fc2_b.reshape(1, -1)]
    full = lambda a: pl.BlockSpec(a.shape, lambda b: (0,) * a.ndim)
    return pl.pallas_call(
        _mega_body,
        grid=(B,),
        in_specs=[pl.BlockSpec((1, N, 128), lambda b: (b, 0, 0))]
        + [full(w) for w in ws],
        out_specs=pl.BlockSpec((1, N, 2), lambda b: (b, 0, 0)),
        out_shape=jax.ShapeDtypeStruct((B, N, 2), jnp.float32),
        scratch_shapes=[pltpu.VMEM((N, N), jnp.float32),
                        pltpu.VMEM((TI * (N + TI) * (N // TI) // 2, 128),
                                   jnp.bfloat16)],
        compiler_params=pltpu.CompilerParams(
            dimension_semantics=("parallel",)),
    )(all_emb, *ws)


# speedup vs baseline: 1.0860x; 1.0260x over previous
import jax
import jax.numpy as jnp
from jax.experimental import pallas as pl
from jax.experimental.pallas import tpu as pltpu

N = 128
NEG = 1e8
TI = 32


def _leaky(x):
    return jnp.maximum(x, 0.01 * x)


def _softmax_rows(x):
    m = jnp.max(x, axis=-1, keepdims=True)
    e = jnp.exp(x - m)
    return e / jnp.sum(e, axis=-1, keepdims=True)


def _DOT(a, b):
    return jnp.dot(a.astype(jnp.bfloat16), b.astype(jnp.bfloat16),
                   preferred_element_type=jnp.float32)


def _edge_sim(x, w0t, w1t, w2t, woutt, b, sp_ref, f_ref, layer):
    D = x.shape[1]
    off = 0
    for t in range(N // TI):
        i0 = t * TI
        jl = N - i0
        rows = TI * jl
        if layer == 0:
            xt = x[i0:i0 + TI]
            xj = x[i0:]
            f = jnp.exp(-jnp.abs(xt[:, None, :] - xj[None, :, :]))
            fb = f.reshape(rows, D).astype(jnp.bfloat16)
            f_ref[off:off + rows] = fb
        else:
            xt = x[i0:i0 + TI, 128:]
            xj = x[i0:, 128:]
            f = jnp.exp(-jnp.abs(xt[:, None, :] - xj[None, :, :]))
            fb = jnp.concatenate(
                [f_ref[off:off + rows],
                 f.reshape(rows, 128).astype(jnp.bfloat16)], axis=-1)
        h = _leaky(_DOT(fb, w0t))
        h = _leaky(_DOT(h, w1t))
        h = _leaky(_DOT(h, w2t))
        s = _DOT(h, woutt).reshape(TI, jl) + b
        sp_ref[i0:i0 + TI, i0:] = s
        sp_ref[i0:, i0:i0 + TI] = s.T
        off += rows


def _mask_aggr(sp, x, nw0t):
    col = jax.lax.broadcasted_iota(jnp.int32, (N, N), 1)
    row = jax.lax.broadcasted_iota(jnp.int32, (N, N), 0)
    eyef = (row == col).astype(jnp.float32)
    sim = _softmax_rows(sp - eyef * NEG)
    dsim = _softmax_rows((1.0 - sim) - eyef * NEG)

    wt = jnp.concatenate([sim.T, dsim.T], axis=1)
    rowj = jax.lax.broadcasted_iota(jnp.int32, (N, 2 * N), 0)
    for _ in range(16):
        mx = jnp.max(wt, axis=0, keepdims=True)
        idx = jnp.min(jnp.where(wt == mx, rowj, N), axis=0, keepdims=True)
        wt = jnp.where(rowj == idx, -1e30, wt)

    mt = (wt < -1e29).astype(jnp.float32)
    m0t = mt[:, :N]
    m1t = mt[:, N:]
    s0 = ((m0t.T + m0t) > 0).astype(jnp.float32)
    s1 = ((m1t.T + m1t) > 0).astype(jnp.float32)
    a0 = _softmax_rows(sim - (1.0 - s0) * NEG)
    a1 = _softmax_rows(dsim - (1.0 - s1) * NEG)
    dm = 1.0 - eyef
    ef0 = a0 * dm
    ef1 = a1 * dm
    ef0 = ef0 / jnp.maximum(jnp.sum(jnp.abs(ef0), -1, keepdims=True), 1e-12)
    ef1 = ef1 / jnp.maximum(jnp.sum(jnp.abs(ef1), -1, keepdims=True), 1e-12)
    ag0 = _DOT(ef0, x)
    ag1 = _DOT(ef1, x)
    nf = jnp.concatenate([x, ag0, ag1], axis=-1)
    return _leaky(_DOT(nf, nw0t))


def _mega_body(x_ref, e0w0_ref, e0w1_ref, e0w2_ref, e0wo_ref, e0b_ref,
               n0_ref, e1w0_ref, e1w1_ref, e1w2_ref, e1wo_ref, e1b_ref,
               n1_ref, fc1_ref, b1_ref, fc2_ref, b2_ref, out_ref, sp_ref,
               f_ref):
    x0 = x_ref[0]
    _edge_sim(x0, e0w0_ref[...], e0w1_ref[...], e0w2_ref[...], e0wo_ref[...],
              e0b_ref[0, 0], sp_ref, f_ref, 0)
    nf0 = _mask_aggr(sp_ref[...], x0, n0_ref[...])
    x1 = jnp.concatenate([x0, nf0], axis=-1)
    _edge_sim(x1, e1w0_ref[...], e1w1_ref[...], e1w2_ref[...], e1wo_ref[...],
              e1b_ref[0, 0], sp_ref, f_ref, 1)
    nf1 = _mask_aggr(sp_ref[...], x1, n1_ref[...])
    x2 = jnp.concatenate([x1, nf1], axis=-1)
    h = _leaky(_DOT(x2, fc1_ref[...]) + b1_ref[...])
    out_ref[0] = _DOT(h, fc2_ref[...]) + b2_ref[...]


def kernel(all_emb, e0_w0, e0_w1, e0_w2, e0_wout, e0_bout, n0_w0, e1_w0,
           e1_w1, e1_w2, e1_wout, e1_bout, n1_w0, fc1_w, fc1_b, fc2_w,
           fc2_b):
    B = all_emb.shape[0]
    ws = [e0_w0.T, e0_w1.T, e0_w2.T, e0_wout.T, e0_bout.reshape(1, 1),
          n0_w0.T, e1_w0.T, e1_w1.T, e1_w2.T, e1_wout.T,
          e1_bout.reshape(1, 1), n1_w0.T, fc1_w.T, fc1_b.reshape(1, -1),
          fc2_w.T, fc2_b.reshape(1, -1)]
    full = lambda a: pl.BlockSpec(a.shape, lambda b: (0,) * a.ndim)
    return pl.pallas_call(
        _mega_body,
        grid=(B,),
        in_specs=[pl.BlockSpec((1, N, 128), lambda b: (b, 0, 0))]
        + [full(w) for w in ws],
        out_specs=pl.BlockSpec((1, N, 2), lambda b: (b, 0, 0)),
        out_shape=jax.ShapeDtypeStruct((B, N, 2), jnp.float32),
        scratch_shapes=[pltpu.VMEM((N, N), jnp.float32),
                        pltpu.VMEM((TI * (N + TI) * (N // TI) // 2, 128),
                                   jnp.bfloat16)],
        compiler_params=pltpu.CompilerParams(
            dimension_semantics=("parallel",)),
    )(all_emb, *ws)
